# baseline (speedup 1.0000x reference)
import functools

import jax
import jax.numpy as jnp
from jax import lax
from jax.experimental import pallas as pl
from jax.experimental.pallas import tpu as pltpu
from jax.experimental.pallas import tpu_sc as plsc

_NUM_CORES = 2
_NUM_SUBCORES = 16
_NUM_WORKERS = _NUM_CORES * _NUM_SUBCORES
_NBUF = 2
_LANES = 16


def _gather_kernel(n_rows, hist, embed_dim, chunk, idx_hbm, table_hbm,
                   out_hbm, idx_v0, idx_v1, rows_v0, rows_v1, t_v0, t_v1,
                   isem, gsem, wsem):
  wid = lax.axis_index("s") * _NUM_CORES + lax.axis_index("c")
  rows_per_w = n_rows // _NUM_WORKERS
  n_chunks = rows_per_w // chunk
  b_per_chunk = chunk // hist
  base = wid * rows_per_w
  b_base = wid * (rows_per_w // hist)
  idx_v = (idx_v0, idx_v1)
  rows_v = (rows_v0, rows_v1)
  t_v = (t_v0, t_v1)

  lane = lax.iota(jnp.int32, _LANES)
  row0 = lane * hist

  def off(i):
    return base + i * chunk

  def start_idx(i, s):
    pltpu.async_copy(idx_hbm.at[pl.ds(off(i), chunk)], idx_v[s],
                     isem.at[s])

  def wait_idx(i, s):
    pltpu.make_async_copy(idx_hbm.at[pl.ds(off(i), chunk)], idx_v[s],
                          isem.at[s]).wait()

  def start_gather(i, s):
    pltpu.async_copy(table_hbm.at[idx_v[s]], rows_v[s], gsem.at[s])

  def wait_gather(i, s):
    pltpu.make_async_copy(table_hbm.at[idx_v[s]], rows_v[s],
                          gsem.at[s]).wait()

  def transpose(s):
    src = rows_v[s]
    dst = t_v[s]

    def h_body(h, carry):
      row_idx = row0 + h
      for e in range(embed_dim):
        col_idx = jnp.full((_LANES,), e, jnp.int32)
        dst[h, e, :] = plsc.load_gather(src, [row_idx, col_idx])
      return carry

    lax.fori_loop(0, hist, h_body, 0)

  def start_wb(i, s):
    b0 = b_base + i * b_per_chunk

    def h_body(h, carry):
      pltpu.async_copy(t_v[s].at[h],
                       out_hbm.at[h, :, pl.ds(b0, b_per_chunk)],
                       wsem.at[s])
      return carry

    lax.fori_loop(0, hist, h_body, 0)

  def wait_wb(i, s):
    b0 = b_base + i * b_per_chunk

    def h_body(h, carry):
      pltpu.make_async_copy(t_v[s].at[h],
                            out_hbm.at[h, :, pl.ds(b0, b_per_chunk)],
                            wsem.at[s]).wait()
      return carry

    lax.fori_loop(0, hist, h_body, 0)

  n_groups = n_chunks // _NBUF

  start_idx(0, 0)
  start_idx(1, 1)
  wait_idx(0, 0)
  start_gather(0, 0)
  for i in (0, 1):
    s = i % _NBUF
    wait_gather(i, s)
    start_idx(i + 2, s)
    wait_idx(i + 1, 1 - s)
    start_gather(i + 1, 1 - s)
    transpose(s)
    start_wb(i, s)

  def group_body(g, carry):
    for s in range(_NBUF):
      i = g * _NBUF + s
      wait_gather(i, s)
      start_idx(i + 2, s)
      wait_idx(i + 1, 1 - s)
      start_gather(i + 1, 1 - s)
      wait_wb(i - 2, s)
      transpose(s)
      start_wb(i, s)
    return carry

  lax.fori_loop(1, n_groups - 1, group_body, 0)

  i = n_chunks - 2
  s = i % _NBUF
  wait_gather(i, s)
  wait_idx(i + 1, 1 - s)
  start_gather(i + 1, 1 - s)
  wait_wb(i - 2, s)
  transpose(s)
  start_wb(i, s)
  i = n_chunks - 1
  s = i % _NBUF
  wait_gather(i, s)
  wait_wb(i - 2, s)
  transpose(s)
  start_wb(i, s)
  wait_wb(n_chunks - 2, (n_chunks - 2) % _NBUF)
  wait_wb(n_chunks - 1, (n_chunks - 1) % _NBUF)


def kernel(indices, table):
  batch, hist = indices.shape
  n_rows = batch * hist
  embed_dim = table.shape[1]
  chunk = _LANES * hist

  flat_idx = indices.reshape(n_rows).astype(jnp.int32)

  mesh = plsc.VectorSubcoreMesh(core_axis_name="c", subcore_axis_name="s")
  k = pl.kernel(
      functools.partial(_gather_kernel, n_rows, hist, embed_dim, chunk),
      out_type=jax.ShapeDtypeStruct((hist, embed_dim, batch), jnp.float32),
      mesh=mesh,
      scratch_types=[
          pltpu.VMEM((chunk,), jnp.int32),
          pltpu.VMEM((chunk,), jnp.int32),
          pltpu.VMEM((chunk, embed_dim), jnp.float32),
          pltpu.VMEM((chunk, embed_dim), jnp.float32),
          pltpu.VMEM((hist, embed_dim, _LANES), jnp.float32),
          pltpu.VMEM((hist, embed_dim, _LANES), jnp.float32),
          pltpu.SemaphoreType.DMA((_NBUF,)),
          pltpu.SemaphoreType.DMA((_NBUF,)),
          pltpu.SemaphoreType.DMA((_NBUF,)),
      ],
      compiler_params=pltpu.CompilerParams(use_tc_tiling_on_sc=False,
                                           needs_layout_passes=False),
  )
  out_heb = k(flat_idx, table)
  return out_heb.transpose(2, 0, 1)


# speedup vs baseline: 1.1110x; 1.1110x over previous
import functools

import jax
import jax.numpy as jnp
from jax import lax
from jax.experimental import pallas as pl
from jax.experimental.pallas import tpu as pltpu
from jax.experimental.pallas import tpu_sc as plsc

_NUM_CORES = 2
_NUM_SUBCORES = 16
_NUM_WORKERS = _NUM_CORES * _NUM_SUBCORES
_NBUF = 2
_LANES = 16
_BBLK = 128
_HBLK = 5


def _gather_kernel(batch, hist, embed_dim, idx_hbm, table_hbm, out_hbm,
                   idxt_v0, idxt_v1, rows_v0, rows_v1, t_v0, t_v1,
                   isem, gsem, wsem):
  wid = lax.axis_index("s") * _NUM_CORES + lax.axis_index("c")
  b_per_w = batch // _NUM_WORKERS
  nbb = b_per_w // _BBLK
  nhg = hist // _HBLK
  n_chunks = nbb * nhg
  bb0 = wid * nbb
  idxt_v = (idxt_v0, idxt_v1)
  rows_v = (rows_v0, rows_v1)
  t_v = (t_v0, t_v1)

  lane = lax.iota(jnp.int32, _LANES)

  def coords(i):
    return bb0 + i % nbb, (i // nbb) * _HBLK

  def start_idx(i, s):
    bb, h0 = coords(i)
    pltpu.async_copy(
        idx_hbm.at[pl.ds(h0, _HBLK), pl.ds(bb * _BBLK, _BBLK)],
        idxt_v[s], isem.at[s])

  def wait_idx(i, s):
    bb, h0 = coords(i)
    pltpu.make_async_copy(
        idx_hbm.at[pl.ds(h0, _HBLK), pl.ds(bb * _BBLK, _BBLK)],
        idxt_v[s], isem.at[s]).wait()

  def start_gather(i, s):
    for hj in range(_HBLK):
      pltpu.async_copy(table_hbm.at[idxt_v[s].at[hj]],
                       rows_v[s].at[pl.ds(hj * _BBLK, _BBLK)], gsem.at[s])

  def wait_gather(i, s):
    for hj in range(_HBLK):
      pltpu.make_async_copy(table_hbm.at[idxt_v[s].at[hj]],
                            rows_v[s].at[pl.ds(hj * _BBLK, _BBLK)],
                            gsem.at[s]).wait()

  def transpose(s):
    src = rows_v[s]
    dst = t_v[s]

    def bib_body(bib, carry):
      rbase = lane + bib * _LANES

      def hj_body(hj, c2):
        row_idx = rbase + hj * _BBLK
        for e in range(embed_dim):
          col_idx = jnp.full((_LANES,), e, jnp.int32)
          dst[hj, e, pl.ds(bib * _LANES, _LANES)] = plsc.load_gather(
              src, [row_idx, col_idx])
        return c2

      lax.fori_loop(0, _HBLK, hj_body, 0)
      return carry

    lax.fori_loop(0, _BBLK // _LANES, bib_body, 0)

  def start_wb(i, s):
    bb, h0 = coords(i)
    for hj in range(_HBLK):
      for eb in range(embed_dim // 8):
        pltpu.async_copy(t_v[s].at[hj, pl.ds(eb * 8, 8), :],
                         out_hbm.at[h0 + hj, eb, bb], wsem.at[s])

  def wait_wb(i, s):
    bb, h0 = coords(i)
    for hj in range(_HBLK):
      for eb in range(embed_dim // 8):
        pltpu.make_async_copy(t_v[s].at[hj, pl.ds(eb * 8, 8), :],
                              out_hbm.at[h0 + hj, eb, bb],
                              wsem.at[s]).wait()

  n_groups = n_chunks // _NBUF

  start_idx(0, 0)
  start_idx(1, 1)
  wait_idx(0, 0)
  start_gather(0, 0)
  for i in (0, 1):
    s = i % _NBUF
    wait_gather(i, s)
    start_idx(i + 2, s)
    wait_idx(i + 1, 1 - s)
    start_gather(i + 1, 1 - s)
    transpose(s)
    start_wb(i, s)

  def group_body(g, carry):
    for s in range(_NBUF):
      i = g * _NBUF + s
      wait_gather(i, s)
      start_idx(i + 2, s)
      wait_idx(i + 1, 1 - s)
      start_gather(i + 1, 1 - s)
      wait_wb(i - 2, s)
      transpose(s)
      start_wb(i, s)
    return carry

  lax.fori_loop(1, n_groups - 1, group_body, 0)

  i = n_chunks - 2
  s = i % _NBUF
  wait_gather(i, s)
  wait_idx(i + 1, 1 - s)
  start_gather(i + 1, 1 - s)
  wait_wb(i - 2, s)
  transpose(s)
  start_wb(i, s)
  i = n_chunks - 1
  s = i % _NBUF
  wait_gather(i, s)
  wait_wb(i - 2, s)
  transpose(s)
  start_wb(i, s)
  wait_wb(n_chunks - 2, (n_chunks - 2) % _NBUF)
  wait_wb(n_chunks - 1, (n_chunks - 1) % _NBUF)


def kernel(indices, table):
  batch, hist = indices.shape
  embed_dim = table.shape[1]
  eb_n = embed_dim // 8
  bb_n = batch // _BBLK

  mesh = plsc.VectorSubcoreMesh(core_axis_name="c", subcore_axis_name="s")
  k = pl.kernel(
      functools.partial(_gather_kernel, batch, hist, embed_dim),
      out_type=jax.ShapeDtypeStruct((hist, eb_n, bb_n, 8, _BBLK),
                                    jnp.float32),
      mesh=mesh,
      scratch_types=[
          pltpu.VMEM((_HBLK, _BBLK), jnp.int32),
          pltpu.VMEM((_HBLK, _BBLK), jnp.int32),
          pltpu.VMEM((_BBLK * _HBLK, embed_dim), jnp.float32),
          pltpu.VMEM((_BBLK * _HBLK, embed_dim), jnp.float32),
          pltpu.VMEM((_HBLK, embed_dim, _BBLK), jnp.float32),
          pltpu.VMEM((_HBLK, embed_dim, _BBLK), jnp.float32),
          pltpu.SemaphoreType.DMA((_NBUF,)),
          pltpu.SemaphoreType.DMA((_NBUF,)),
          pltpu.SemaphoreType.DMA((_NBUF,)),
      ],
      compiler_params=pltpu.CompilerParams(use_tc_tiling_on_sc=False,
                                           needs_layout_passes=False),
  )
  out5 = k(indices.T.astype(jnp.int32), table)
  return out5.transpose(2, 4, 0, 1, 3).reshape(batch, hist, embed_dim)


# speedup vs baseline: 1.1132x; 1.0020x over previous
import functools

import jax
import jax.numpy as jnp
from jax import lax
from jax.experimental import pallas as pl
from jax.experimental.pallas import tpu as pltpu
from jax.experimental.pallas import tpu_sc as plsc

_NUM_CORES = 2
_NUM_SUBCORES = 16
_NUM_WORKERS = _NUM_CORES * _NUM_SUBCORES
_NBUF = 2
_LANES = 16
_BBLK = 128
_HBLK = 5


def _gather_kernel(batch, hist, embed_dim, idx_hbm, table_hbm, out_hbm,
                   idxt_v0, idxt_v1, rows_v0, rows_v1, t_v0, t_v1,
                   isem, gsem, wsem):
  wid = lax.axis_index("s") * _NUM_CORES + lax.axis_index("c")
  b_per_w = batch // _NUM_WORKERS
  nbb = b_per_w // _BBLK
  nhg = hist // _HBLK
  n_chunks = nbb * nhg
  bb0 = wid * nbb
  idxt_v = (idxt_v0, idxt_v1)
  rows_v = (rows_v0, rows_v1)
  t_v = (t_v0, t_v1)

  lane = lax.iota(jnp.int32, _LANES)

  def coords(i):
    return bb0 + i % nbb, (i // nbb) * _HBLK

  def start_idx(i, s):
    bb, h0 = coords(i)
    pltpu.async_copy(
        idx_hbm.at[pl.ds(h0, _HBLK), pl.ds(bb * _BBLK, _BBLK)],
        idxt_v[s], isem.at[s])

  def wait_idx(i, s):
    bb, h0 = coords(i)
    pltpu.make_async_copy(
        idx_hbm.at[pl.ds(h0, _HBLK), pl.ds(bb * _BBLK, _BBLK)],
        idxt_v[s], isem.at[s]).wait()

  def start_gather(i, s):
    for hj in range(_HBLK):
      pltpu.async_copy(table_hbm.at[idxt_v[s].at[hj]],
                       rows_v[s].at[pl.ds(hj * _BBLK, _BBLK)], gsem.at[s])

  def wait_gather(i, s):
    for hj in range(_HBLK):
      pltpu.make_async_copy(table_hbm.at[idxt_v[s].at[hj]],
                            rows_v[s].at[pl.ds(hj * _BBLK, _BBLK)],
                            gsem.at[s]).wait()

  def transpose(s):
    src = rows_v[s]
    dst = t_v[s]

    row_idxs = [lane + bib * _LANES + hj * _BBLK
                for bib in range(_BBLK // _LANES) for hj in range(_HBLK)]

    def e_body(e, carry):
      col_idx = jnp.full((_LANES,), e, jnp.int32)
      n = 0
      for bib in range(_BBLK // _LANES):
        for hj in range(_HBLK):
          dst[hj, e, pl.ds(bib * _LANES, _LANES)] = plsc.load_gather(
              src, [row_idxs[n], col_idx])
          n += 1
      return carry

    lax.fori_loop(0, embed_dim, e_body, 0)

  def start_wb(i, s):
    bb, h0 = coords(i)
    for hj in range(_HBLK):
      for eb in range(embed_dim // 8):
        pltpu.async_copy(t_v[s].at[hj, pl.ds(eb * 8, 8), :],
                         out_hbm.at[h0 + hj, eb, bb], wsem.at[s])

  def wait_wb(i, s):
    bb, h0 = coords(i)
    for hj in range(_HBLK):
      for eb in range(embed_dim // 8):
        pltpu.make_async_copy(t_v[s].at[hj, pl.ds(eb * 8, 8), :],
                              out_hbm.at[h0 + hj, eb, bb],
                              wsem.at[s]).wait()

  n_groups = n_chunks // _NBUF

  start_idx(0, 0)
  start_idx(1, 1)
  wait_idx(0, 0)
  start_gather(0, 0)
  for i in (0, 1):
    s = i % _NBUF
    wait_gather(i, s)
    start_idx(i + 2, s)
    wait_idx(i + 1, 1 - s)
    start_gather(i + 1, 1 - s)
    transpose(s)
    start_wb(i, s)

  def group_body(g, carry):
    for s in range(_NBUF):
      i = g * _NBUF + s
      wait_gather(i, s)
      start_idx(i + 2, s)
      wait_idx(i + 1, 1 - s)
      start_gather(i + 1, 1 - s)
      wait_wb(i - 2, s)
      transpose(s)
      start_wb(i, s)
    return carry

  lax.fori_loop(1, n_groups - 1, group_body, 0)

  i = n_chunks - 2
  s = i % _NBUF
  wait_gather(i, s)
  wait_idx(i + 1, 1 - s)
  start_gather(i + 1, 1 - s)
  wait_wb(i - 2, s)
  transpose(s)
  start_wb(i, s)
  i = n_chunks - 1
  s = i % _NBUF
  wait_gather(i, s)
  wait_wb(i - 2, s)
  transpose(s)
  start_wb(i, s)
  wait_wb(n_chunks - 2, (n_chunks - 2) % _NBUF)
  wait_wb(n_chunks - 1, (n_chunks - 1) % _NBUF)


def kernel(indices, table):
  batch, hist = indices.shape
  embed_dim = table.shape[1]
  eb_n = embed_dim // 8
  bb_n = batch // _BBLK

  mesh = plsc.VectorSubcoreMesh(core_axis_name="c", subcore_axis_name="s")
  k = pl.kernel(
      functools.partial(_gather_kernel, batch, hist, embed_dim),
      out_type=jax.ShapeDtypeStruct((hist, eb_n, bb_n, 8, _BBLK),
                                    jnp.float32),
      mesh=mesh,
      scratch_types=[
          pltpu.VMEM((_HBLK, _BBLK), jnp.int32),
          pltpu.VMEM((_HBLK, _BBLK), jnp.int32),
          pltpu.VMEM((_BBLK * _HBLK, embed_dim), jnp.float32),
          pltpu.VMEM((_BBLK * _HBLK, embed_dim), jnp.float32),
          pltpu.VMEM((_HBLK, embed_dim, _BBLK), jnp.float32),
          pltpu.VMEM((_HBLK, embed_dim, _BBLK), jnp.float32),
          pltpu.SemaphoreType.DMA((_NBUF,)),
          pltpu.SemaphoreType.DMA((_NBUF,)),
          pltpu.SemaphoreType.DMA((_NBUF,)),
      ],
      compiler_params=pltpu.CompilerParams(use_tc_tiling_on_sc=False,
                                           needs_layout_passes=False),
  )
  out5 = k(indices.T.astype(jnp.int32), table)
  return out5.transpose(2, 4, 0, 1, 3).reshape(batch, hist, embed_dim)


# speedup vs baseline: 1.7507x; 1.5727x over previous
import functools

import jax
import jax.numpy as jnp
from jax import lax
from jax.experimental import pallas as pl
from jax.experimental.pallas import tpu as pltpu
from jax.experimental.pallas import tpu_sc as plsc

_NUM_CORES = 2
_NUM_SUBCORES = 16
_NUM_WORKERS = _NUM_CORES * _NUM_SUBCORES
_NBUF = 2
_LANES = 16
_BBLK = 128
_HBLK = 5


def _gather_kernel(batch, hist, embed_dim, idx_hbm, table_hbm, out_hbm,
                   idxt_v0, idxt_v1, rows_v0, rows_v1, t_v0, t_v1,
                   isem, gsem, wsem):
  wid = lax.axis_index("s") * _NUM_CORES + lax.axis_index("c")
  b_per_w = batch // _NUM_WORKERS
  nbb = b_per_w // _BBLK
  nhg = hist // _HBLK
  n_chunks = nbb * nhg
  bb0 = wid * nbb
  idxt_v = (idxt_v0, idxt_v1)
  rows_v = (rows_v0, rows_v1)
  t_v = (t_v0, t_v1)

  lane = lax.iota(jnp.int32, _LANES)

  def coords(i):
    return bb0 + i % nbb, (i // nbb) * _HBLK

  def start_idx(i, s):
    bb, h0 = coords(i)
    for hj in range(_HBLK):
      pltpu.async_copy(
          idx_hbm.at[h0 + hj, pl.ds(bb * _BBLK, _BBLK)],
          idxt_v[s].at[pl.ds(hj * _BBLK, _BBLK)], isem.at[s])

  def wait_idx(i, s):
    bb, h0 = coords(i)
    for hj in range(_HBLK):
      pltpu.make_async_copy(
          idx_hbm.at[h0 + hj, pl.ds(bb * _BBLK, _BBLK)],
          idxt_v[s].at[pl.ds(hj * _BBLK, _BBLK)], isem.at[s]).wait()

  def start_gather(i, s):
    pltpu.async_copy(table_hbm.at[idxt_v[s]], rows_v[s], gsem.at[s])

  def wait_gather(i, s):
    pltpu.make_async_copy(table_hbm.at[idxt_v[s]], rows_v[s],
                          gsem.at[s]).wait()

  def transpose(s):
    src = rows_v[s]
    dst = t_v[s]
    row_consts = [lane + hj * embed_dim + k * _LANES
                  for hj in range(_HBLK) for k in range(embed_dim // _LANES)]

    def bi_body(bi, carry):
      col_idx = jnp.full((_LANES,), bi, jnp.int32)
      n = 0
      for hj in range(_HBLK):
        for k in range(embed_dim // _LANES):
          v = src[hj * _BBLK + bi, pl.ds(k * _LANES, _LANES)]
          plsc.store_scatter(dst, [row_consts[n], col_idx], v)
          n += 1
      return carry

    lax.fori_loop(0, _BBLK, bi_body, 0)

  def start_wb(i, s):
    bb, h0 = coords(i)
    for hj in range(_HBLK):
      for eb in range(embed_dim // 8):
        pltpu.async_copy(
            t_v[s].at[pl.ds(hj * embed_dim + eb * 8, 8), pl.ds(0, _BBLK)],
            out_hbm.at[h0 + hj, eb, bb], wsem.at[s])

  def wait_wb(i, s):
    bb, h0 = coords(i)
    for hj in range(_HBLK):
      for eb in range(embed_dim // 8):
        pltpu.make_async_copy(
            t_v[s].at[pl.ds(hj * embed_dim + eb * 8, 8), pl.ds(0, _BBLK)],
            out_hbm.at[h0 + hj, eb, bb], wsem.at[s]).wait()

  n_groups = n_chunks // _NBUF

  start_idx(0, 0)
  start_idx(1, 1)
  wait_idx(0, 0)
  start_gather(0, 0)
  for i in (0, 1):
    s = i % _NBUF
    wait_gather(i, s)
    start_idx(i + 2, s)
    wait_idx(i + 1, 1 - s)
    start_gather(i + 1, 1 - s)
    transpose(s)
    start_wb(i, s)

  def group_body(g, carry):
    for s in range(_NBUF):
      i = g * _NBUF + s
      wait_gather(i, s)
      start_idx(i + 2, s)
      wait_idx(i + 1, 1 - s)
      start_gather(i + 1, 1 - s)
      wait_wb(i - 2, s)
      transpose(s)
      start_wb(i, s)
    return carry

  lax.fori_loop(1, n_groups - 1, group_body, 0)

  i = n_chunks - 2
  s = i % _NBUF
  wait_gather(i, s)
  wait_idx(i + 1, 1 - s)
  start_gather(i + 1, 1 - s)
  wait_wb(i - 2, s)
  transpose(s)
  start_wb(i, s)
  i = n_chunks - 1
  s = i % _NBUF
  wait_gather(i, s)
  wait_wb(i - 2, s)
  transpose(s)
  start_wb(i, s)
  wait_wb(n_chunks - 2, (n_chunks - 2) % _NBUF)
  wait_wb(n_chunks - 1, (n_chunks - 1) % _NBUF)


def kernel(indices, table):
  batch, hist = indices.shape
  embed_dim = table.shape[1]
  eb_n = embed_dim // 8
  bb_n = batch // _BBLK

  mesh = plsc.VectorSubcoreMesh(core_axis_name="c", subcore_axis_name="s")
  k = pl.kernel(
      functools.partial(_gather_kernel, batch, hist, embed_dim),
      out_type=jax.ShapeDtypeStruct((hist, eb_n, bb_n, 8, _BBLK),
                                    jnp.float32),
      mesh=mesh,
      scratch_types=[
          pltpu.VMEM((_HBLK * _BBLK,), jnp.int32),
          pltpu.VMEM((_HBLK * _BBLK,), jnp.int32),
          pltpu.VMEM((_BBLK * _HBLK, embed_dim), jnp.float32),
          pltpu.VMEM((_BBLK * _HBLK, embed_dim), jnp.float32),
          pltpu.VMEM((_HBLK * embed_dim, _BBLK + 1), jnp.float32),
          pltpu.VMEM((_HBLK * embed_dim, _BBLK + 1), jnp.float32),
          pltpu.SemaphoreType.DMA((_NBUF,)),
          pltpu.SemaphoreType.DMA((_NBUF,)),
          pltpu.SemaphoreType.DMA((_NBUF,)),
      ],
      compiler_params=pltpu.CompilerParams(use_tc_tiling_on_sc=False,
                                           needs_layout_passes=False),
  )
  out5 = k(indices.T.astype(jnp.int32), table)
  return out5.transpose(2, 4, 0, 1, 3).reshape(batch, hist, embed_dim)
